# Initial kernel scaffold; baseline (speedup 1.0000x reference)
import jax
import jax.numpy as jnp
from jax.experimental import pallas as pl


def kernel(x, edge_index, batch, params):
    raise NotImplementedError("write your pallas kernel here")


# speedup vs baseline: 2.0853x; 2.0853x over previous
import functools
import math

import jax
import jax.numpy as jnp
from jax import lax
from jax.experimental import pallas as pl
from jax.experimental.pallas import tpu as pltpu

D = 128
B = 40
MAXN = 512
H = 4
DH = D // H
K1 = 75
K1P = 80
BK = 512
NEG = -1e9
_INV_SQRT_D = 1.0 / math.sqrt(D)


def _relu(v):
    return jnp.maximum(v, 0.0)


def _mm(a, b):
    return jnp.dot(a, b, preferred_element_type=jnp.float32)


def _mlp_head_body(x_ref, w0, b0, w1, b1, w2, b2, w3, b3, t_ref):
    h = x_ref[...]
    for w, b in ((w0, b0), (w1, b1), (w2, b2), (w3, b3)):
        h = _relu(_mm(h, w[...]) + b[...])
    t_ref[...] = _relu(_mm(h, w1[...]) + b1[...])


def _combine_body(p0, p1, t, dg0, dg1, wl, bl, wr, wn, bn, t_ref):
    deg = dg0[...] + dg1[...]
    agg = (p0[...] + p1[...]) * (1.0 / jnp.maximum(deg, 1.0))
    u = _relu(_mm(agg, wl[...]) + bl[...] + _mm(t[...], wr[...]))
    t_ref[...] = _relu(_mm(u, wn[...]) + bn[...])


def _final_body(p0, p1, t, dg0, dg1, wl, bl, wr, pw, pb, kw, vw,
                yk_ref, yv_ref, c_ref):
    deg = dg0[...] + dg1[...]
    agg = (p0[...] + p1[...]) * (1.0 / jnp.maximum(deg, 1.0))
    u = _relu(_mm(agg, wl[...]) + bl[...] + _mm(t[...], wr[...]))
    xf = _mm(u, pw[...]) + pb[...]
    c = lax.rsqrt(deg + 1.0)
    yk_ref[...] = _mm(xf, kw[...]) * c
    yv_ref[...] = _mm(xf, vw[...]) * c
    c_ref[...] = c


def _kv_finish_body(pk0, pk1, yk, pv0, pv1, yv, c, bk, bv, kf_ref, vf_ref):
    cc = c[...]
    kf_ref[...] = (pk0[...] + pk1[...] + yk[...]) * cc + bk[...]
    vf_ref[...] = (pv0[...] + pv1[...] + yv[...]) * cc + bv[...]


def _row_specs(n, shapes):
    specs = []
    for s in shapes:
        if s == "rows":
            specs.append(pl.BlockSpec((BK, D), lambda i: (i, 0)))
        elif s == "col":
            specs.append(pl.BlockSpec((BK, 1), lambda i: (i, 0)))
        else:
            specs.append(pl.BlockSpec(s, lambda i: (0,) * len(s)))
    return specs


def _call_rows(body, np_, in_arrays, in_kinds, n_row_outs, n_col_outs=0):
    grid = (np_ // BK,)
    out_shapes = ([jax.ShapeDtypeStruct((np_, D), jnp.float32)] * n_row_outs
                  + [jax.ShapeDtypeStruct((np_, 1), jnp.float32)] * n_col_outs)
    out_specs = ([pl.BlockSpec((BK, D), lambda i: (i, 0))] * n_row_outs
                 + [pl.BlockSpec((BK, 1), lambda i: (i, 0))] * n_col_outs)
    return pl.pallas_call(
        body,
        grid=grid,
        in_specs=_row_specs(np_, in_kinds),
        out_specs=out_specs if len(out_specs) > 1 else out_specs[0],
        out_shape=out_shapes if len(out_shapes) > 1 else out_shapes[0],
    )(*in_arrays)


def _mab(q_in, kd, vd, wq, bq, wo, bo, maskrow):
    qp = _mm(q_in, wq) + bq
    outs = []
    for h in range(H):
        q = qp[:, h * DH:(h + 1) * DH]
        k = kd[:, h * DH:(h + 1) * DH]
        v = vd[:, h * DH:(h + 1) * DH]
        att = lax.dot_general(q, k, (((1,), (1,)), ((), ()))) * _INV_SQRT_D
        att = att + maskrow
        m = jnp.max(att, axis=-1, keepdims=True)
        e = jnp.exp(att - m)
        a = e / jnp.sum(e, axis=-1, keepdims=True)
        outs.append(q + _mm(a, v))
    o = jnp.concatenate(outs, axis=1)
    return o + _relu(_mm(o, wo) + bo)


def _attn_body(count_ref, kd_ref, vd_ref, s1, fcqw, fcqb, ow, ob,
               skw, skb, svw, svb, sfw, sfb, sow, sob,
               pkw, pkb, pvw, pvb, pfw, pfb, pow_, pob, s2, w2, b2,
               out_ref):
    b = pl.program_id(0)
    cnt = count_ref[b]
    kd = kd_ref[0]
    vd = vd_ref[0]
    col = lax.broadcasted_iota(jnp.int32, (1, MAXN), 1)
    mask1 = jnp.where(col >= cnt, NEG, 0.0)
    rowi = lax.broadcasted_iota(jnp.int32, (MAXN, 1), 0)
    vd = jnp.where(rowi < cnt, vd, 0.0)

    bx = _mab(s1[...], kd, vd, fcqw[...], fcqb[...], ow[...], ob[...], mask1)

    colp = lax.broadcasted_iota(jnp.int32, (1, K1P), 1)
    maskp = jnp.where(colp >= K1, NEG, 0.0)
    kd2 = _mm(bx, skw[...]) + skb[...]
    vd2 = _mm(bx, svw[...]) + svb[...]
    bx = _mab(bx, kd2, vd2, sfw[...], sfb[...], sow[...], sob[...], maskp)

    kd3 = _mm(bx, pkw[...]) + pkb[...]
    vd3 = _mm(bx, pvw[...]) + pvb[...]
    bx3 = _mab(s2[...], kd3, vd3, pfw[...], pfb[...], pow_[...], pob[...], maskp)

    res = jnp.sum(bx3[0:1, :] * w2[...]) + b2[0, 0]
    out_ref[...] = jnp.broadcast_to(res, (1, 8, D))


def _attention(counts, kdense, vdense, p):
    s1 = jnp.zeros((K1P, D), jnp.float32).at[:K1].set(p['pma1_S'])
    s2 = jnp.zeros((8, D), jnp.float32).at[0:1].set(p['pma2_S'])
    w2 = p['p_lin2_W'].T
    b2 = p['p_lin2_b'].reshape(1, 1)

    def wspec(shape):
        return pl.BlockSpec(shape, lambda b, c: (0,) * len(shape))

    ins = [kdense, vdense, s1,
           p['pma1_fcq_W'], p['pma1_fcq_b'][None, :],
           p['pma1_o_W'], p['pma1_o_b'][None, :],
           p['sab_k_W'], p['sab_k_b'][None, :],
           p['sab_v_W'], p['sab_v_b'][None, :],
           p['sab_fcq_W'], p['sab_fcq_b'][None, :],
           p['sab_o_W'], p['sab_o_b'][None, :],
           p['pma2_k_W'], p['pma2_k_b'][None, :],
           p['pma2_v_W'], p['pma2_v_b'][None, :],
           p['pma2_fcq_W'], p['pma2_fcq_b'][None, :],
           p['pma2_o_W'], p['pma2_o_b'][None, :],
           s2, w2, b2]
    in_specs = [pl.BlockSpec((1, MAXN, D), lambda b, c: (b, 0, 0)),
                pl.BlockSpec((1, MAXN, D), lambda b, c: (b, 0, 0))]
    for a in ins[2:]:
        in_specs.append(wspec(a.shape))

    grid_spec = pltpu.PrefetchScalarGridSpec(
        num_scalar_prefetch=1,
        grid=(B,),
        in_specs=in_specs,
        out_specs=pl.BlockSpec((1, 8, D), lambda b, c: (b, 0, 0)),
    )
    out = pl.pallas_call(
        _attn_body,
        grid_spec=grid_spec,
        out_shape=jax.ShapeDtypeStruct((B, 8, D), jnp.float32),
    )(counts, *ins)
    return out[:, 0, 0]


def _message(feat_pad, src, dst, n, np_):
    s = feat_pad[src]
    agg = jax.ops.segment_sum(s, dst, num_segments=n)
    agg = jnp.zeros((np_, D), jnp.float32).at[:n].set(agg)
    return agg, jnp.zeros((np_, D), jnp.float32)


def _degree(dst, n, np_):
    deg = jnp.bincount(dst, length=n).astype(jnp.float32)
    deg = jnp.zeros((np_,), jnp.float32).at[:n].set(deg)
    return deg[:, None], jnp.zeros((np_, 1), jnp.float32)


def _gather_rows(feat_pad, widx):
    return feat_pad[widx]


def kernel(x, edge_index, batch, params):
    n, d = x.shape
    p = params
    np_ = ((n + MAXN + BK - 1) // BK) * BK
    src = edge_index[0].astype(jnp.int32)
    dst = edge_index[1].astype(jnp.int32)
    batch = batch.astype(jnp.int32)

    x_pad = jnp.zeros((np_, d), jnp.float32).at[:n].set(x)
    dg0, dg1 = _degree(dst, n, np_)

    bias = lambda k: p[k][None, :]

    t = _call_rows(
        _mlp_head_body, np_,
        [x_pad, p['in_W'], bias('in_b'), p['l1_W'], bias('l1_b'),
         p['l2_W'], bias('l2_b'), p['l3_W'], bias('l3_b')],
        ["rows", (D, D), (1, D), (D, D), (1, D), (D, D), (1, D), (D, D), (1, D)],
        n_row_outs=1)

    for j in (1, 2, 3):
        p0, p1 = _message(t, src, dst, n, np_)
        if j < 3:
            nw, nb = p[f'l{j + 1}_W'], bias(f'l{j + 1}_b')
            t = _call_rows(
                _combine_body, np_,
                [p0, p1, t, dg0, dg1, p[f'c{j}_Wl'], bias(f'c{j}_bl'),
                 p[f'c{j}_Wr'], nw, nb],
                ["rows", "rows", "rows", "col", "col",
                 (D, D), (1, D), (D, D), (D, D), (1, D)],
                n_row_outs=1)
        else:
            yk, yv, c = _call_rows(
                _final_body, np_,
                [p0, p1, t, dg0, dg1, p['c3_Wl'], bias('c3_bl'), p['c3_Wr'],
                 p['p_lin1_W'], bias('p_lin1_b'), p['pma1_k_W'], p['pma1_v_W']],
                ["rows", "rows", "rows", "col", "col",
                 (D, D), (1, D), (D, D), (D, D), (1, D), (D, D), (D, D)],
                n_row_outs=2, n_col_outs=1)

    pk0, pk1 = _message(yk, src, dst, n, np_)
    pv0, pv1 = _message(yv, src, dst, n, np_)
    kf, vf = _call_rows(
        _kv_finish_body, np_,
        [pk0, pk1, yk, pv0, pv1, yv, c, bias('pma1_k_b'), bias('pma1_v_b')],
        ["rows", "rows", "rows", "rows", "rows", "rows", "col", (1, D), (1, D)],
        n_row_outs=2)

    ptr = jnp.searchsorted(batch, jnp.arange(B, dtype=jnp.int32)).astype(jnp.int32)
    counts = jnp.diff(jnp.concatenate([ptr, jnp.array([n], jnp.int32)]))
    widx = ptr[:, None] + jnp.arange(MAXN, dtype=jnp.int32)[None, :]
    kdense = _gather_rows(kf, widx)
    vdense = _gather_rows(vf, widx)

    return _attention(counts, kdense, vdense, p)


# speedup vs baseline: 6.5261x; 3.1295x over previous
import functools
import math

import jax
import jax.numpy as jnp
from jax import lax
from jax.experimental import pallas as pl
from jax.experimental.pallas import tpu as pltpu
from jax.experimental.pallas import tpu_sc as plsc

D = 128
B = 40
MAXN = 512
H = 4
DH = D // H
K1 = 75
K1P = 80
WIN = MAXN + 8
BK = 512
NEG = -1e9
_INV_SQRT_D = 1.0 / math.sqrt(D)


def _relu(v):
    return jnp.maximum(v, 0.0)


def _mm(a, b):
    return jnp.dot(a, b, preferred_element_type=jnp.float32)


def _mlp_head_body(x_ref, w0, b0, w1, b1, w2, b2, w3, b3, t_ref):
    h = x_ref[...]
    for w, b in ((w0, b0), (w1, b1), (w2, b2), (w3, b3)):
        h = _relu(_mm(h, w[...]) + b[...])
    t_ref[...] = _relu(_mm(h, w1[...]) + b1[...])


def _combine_body(p0, p1, t, dg0, wl, bl, wr, wn, bn, t_ref):
    deg = dg0[...]
    agg = jnp.concatenate([p0[...], p1[...]], axis=1) \
        * (1.0 / jnp.maximum(deg, 1.0))
    u = _relu(_mm(agg, wl[...]) + bl[...] + _mm(t[...], wr[...]))
    t_ref[...] = _relu(_mm(u, wn[...]) + bn[...])


def _final_body(p0, p1, t, dg0, wl, bl, wr, pw, pb, kw, vw,
                yk_ref, yv_ref, c_ref):
    deg = dg0[...]
    agg = jnp.concatenate([p0[...], p1[...]], axis=1) \
        * (1.0 / jnp.maximum(deg, 1.0))
    u = _relu(_mm(agg, wl[...]) + bl[...] + _mm(t[...], wr[...]))
    xf = _mm(u, pw[...]) + pb[...]
    c = lax.rsqrt(deg + 1.0)
    yk_ref[...] = _mm(xf, kw[...]) * c
    yv_ref[...] = _mm(xf, vw[...]) * c
    c_ref[...] = c


def _kv_finish_body(pk0, pk1, yk, pv0, pv1, yv, c, bk, bv, kf_ref, vf_ref):
    cc = c[...]
    pk = jnp.concatenate([pk0[...], pk1[...]], axis=1)
    pv = jnp.concatenate([pv0[...], pv1[...]], axis=1)
    kf_ref[...] = (pk + yk[...]) * cc + bk[...]
    vf_ref[...] = (pv + yv[...]) * cc + bv[...]


def _row_specs(n, shapes):
    specs = []
    for s in shapes:
        if s == "rows":
            specs.append(pl.BlockSpec((BK, D), lambda i: (i, 0)))
        elif s == "col":
            specs.append(pl.BlockSpec((BK, 1), lambda i: (i, 0)))
        elif s == "half":
            specs.append(pl.BlockSpec((BK, HD), lambda i: (i, 0)))
        else:
            specs.append(pl.BlockSpec(s, lambda i: (0,) * len(s)))
    return specs


def _call_rows(body, np_, in_arrays, in_kinds, n_row_outs, n_col_outs=0):
    grid = (np_ // BK,)
    out_shapes = ([jax.ShapeDtypeStruct((np_, D), jnp.float32)] * n_row_outs
                  + [jax.ShapeDtypeStruct((np_, 1), jnp.float32)] * n_col_outs)
    out_specs = ([pl.BlockSpec((BK, D), lambda i: (i, 0))] * n_row_outs
                 + [pl.BlockSpec((BK, 1), lambda i: (i, 0))] * n_col_outs)
    return pl.pallas_call(
        body,
        grid=grid,
        in_specs=_row_specs(np_, in_kinds),
        out_specs=out_specs if len(out_specs) > 1 else out_specs[0],
        out_shape=out_shapes if len(out_shapes) > 1 else out_shapes[0],
    )(*in_arrays)


def _mab(q_in, kd, vd, wq, bq, wo, bo, maskrow):
    qp = _mm(q_in, wq) + bq
    outs = []
    for h in range(H):
        q = qp[:, h * DH:(h + 1) * DH]
        k = kd[:, h * DH:(h + 1) * DH]
        v = vd[:, h * DH:(h + 1) * DH]
        att = lax.dot_general(q, k, (((1,), (1,)), ((), ()))) * _INV_SQRT_D
        att = att + maskrow
        m = jnp.max(att, axis=-1, keepdims=True)
        e = jnp.exp(att - m)
        a = e / jnp.sum(e, axis=-1, keepdims=True)
        outs.append(q + _mm(a, v))
    o = jnp.concatenate(outs, axis=1)
    return o + _relu(_mm(o, wo) + bo)


def _attn_body(count_ref, ptr_ref, kf_hbm, vf_hbm, s1, fcqw, fcqb, ow, ob,
               skw, skb, svw, svb, sfw, sfb, sow, sob,
               pkw, pkb, pvw, pvb, pfw, pfb, pow_, pob, s2, w2, b2,
               out_ref, kbuf, vbuf, sem0, sem1):
    b = pl.program_id(0)
    cnt = jnp.minimum(count_ref[b], MAXN)
    start = ptr_ref[b]
    start8 = (start // 8) * 8
    shift = start - start8
    cpk = pltpu.make_async_copy(kf_hbm.at[pl.ds(start8, WIN)], kbuf, sem0)
    cpv = pltpu.make_async_copy(vf_hbm.at[pl.ds(start8, WIN)], vbuf, sem1)
    cpk.start()
    cpv.start()
    cpk.wait()
    cpv.wait()
    col = lax.broadcasted_iota(jnp.int32, (1, WIN), 1)
    mask1 = jnp.where((col >= shift) & (col < shift + cnt), 0.0, NEG)
    rowi = lax.broadcasted_iota(jnp.int32, (WIN, 1), 0)
    rvalid = (rowi >= shift) & (rowi < shift + cnt)
    kd = jnp.where(rvalid, kbuf[...], 0.0)
    vd = jnp.where(rvalid, vbuf[...], 0.0)

    bx = _mab(s1[...], kd, vd, fcqw[...], fcqb[...], ow[...], ob[...], mask1)

    colp = lax.broadcasted_iota(jnp.int32, (1, K1P), 1)
    maskp = jnp.where(colp >= K1, NEG, 0.0)
    kd2 = _mm(bx, skw[...]) + skb[...]
    vd2 = _mm(bx, svw[...]) + svb[...]
    bx = _mab(bx, kd2, vd2, sfw[...], sfb[...], sow[...], sob[...], maskp)

    kd3 = _mm(bx, pkw[...]) + pkb[...]
    vd3 = _mm(bx, pvw[...]) + pvb[...]
    bx3 = _mab(s2[...], kd3, vd3, pfw[...], pfb[...], pow_[...], pob[...], maskp)

    res = jnp.sum(bx3[0:1, :] * w2[...]) + b2[0, 0]
    out_ref[...] = jnp.broadcast_to(res, (1, 8, D))


def _attention(counts, ptr, kf, vf, p):
    s1 = jnp.zeros((K1P, D), jnp.float32).at[:K1].set(p['pma1_S'])
    s2 = jnp.zeros((8, D), jnp.float32).at[0:1].set(p['pma2_S'])
    w2 = p['p_lin2_W'].T
    b2 = p['p_lin2_b'].reshape(1, 1)

    def wspec(shape):
        return pl.BlockSpec(shape, lambda b, c, q: (0,) * len(shape))

    ins = [kf, vf, s1,
           p['pma1_fcq_W'], p['pma1_fcq_b'][None, :],
           p['pma1_o_W'], p['pma1_o_b'][None, :],
           p['sab_k_W'], p['sab_k_b'][None, :],
           p['sab_v_W'], p['sab_v_b'][None, :],
           p['sab_fcq_W'], p['sab_fcq_b'][None, :],
           p['sab_o_W'], p['sab_o_b'][None, :],
           p['pma2_k_W'], p['pma2_k_b'][None, :],
           p['pma2_v_W'], p['pma2_v_b'][None, :],
           p['pma2_fcq_W'], p['pma2_fcq_b'][None, :],
           p['pma2_o_W'], p['pma2_o_b'][None, :],
           s2, w2, b2]
    in_specs = [pl.BlockSpec(memory_space=pl.ANY),
                pl.BlockSpec(memory_space=pl.ANY)]
    for a in ins[2:]:
        in_specs.append(wspec(a.shape))

    grid_spec = pltpu.PrefetchScalarGridSpec(
        num_scalar_prefetch=2,
        grid=(B,),
        in_specs=in_specs,
        out_specs=pl.BlockSpec((1, 8, D), lambda b, c, q: (b, 0, 0)),
        scratch_shapes=[pltpu.VMEM((WIN, D), jnp.float32),
                        pltpu.VMEM((WIN, D), jnp.float32),
                        pltpu.SemaphoreType.DMA,
                        pltpu.SemaphoreType.DMA],
    )
    out = pl.pallas_call(
        _attn_body,
        grid_spec=grid_spec,
        out_shape=jax.ShapeDtypeStruct((B, 8, D), jnp.float32),
    )(counts, ptr, *ins)
    return out[:, 0, 0]


NC = 2
NS = 16
TILES = NC * NS
CH = 128


HD = D // 2


def _build_msg_kernel(np_, nacc, nch):
    mesh = plsc.VectorSubcoreMesh(core_axis_name="c", subcore_axis_name="s")
    rows_sub = nacc // NS
    out_types = jax.ShapeDtypeStruct((NC, np_, HD), jnp.float32)
    scratch = [
        pltpu.VMEM((nch + 1, CH), jnp.int32),
        pltpu.VMEM((nch, CH), jnp.int32),
        pltpu.VMEM((CH, HD), jnp.float32),
        pltpu.VMEM((CH, HD), jnp.float32),
        pltpu.VMEM_SHARED((nacc, HD), jnp.float32),
        pltpu.SemaphoreType.DMA,
        pltpu.SemaphoreType.DMA,
    ]

    def body(feat, srcp, dstp, out, src_v, dst_v, b0, b1, acc, sem0, sem1):
        c = lax.axis_index("c")
        s = lax.axis_index("s")
        pltpu.sync_copy(srcp.at[s], src_v)
        pltpu.sync_copy(dstp.at[s], dst_v)

        zero16 = jnp.zeros((16,), jnp.float32)

        @pl.when(c == 1)
        def _():
            def addc(i, carry):
                for k in range(CH // 16):
                    sl = pl.ds(k * 16, 16)
                    src_v[i, sl] = src_v[i, sl] + 1
                return carry
            lax.fori_loop(0, nch + 1, addc, 0)

        def zf(i, carry):
            for k in range(HD // 16):
                b0[i, pl.ds(k * 16, 16)] = zero16
            return carry
        lax.fori_loop(0, CH, zf, 0)

        off = s * rows_sub
        nfull, rem = rows_sub // CH, rows_sub % CH
        for m in range(nfull):
            pltpu.sync_copy(b0, acc.at[pl.ds(off + m * CH, CH)])
        if rem:
            pltpu.sync_copy(b0.at[pl.ds(0, rem)],
                            acc.at[pl.ds(off + nfull * CH, rem)])
        plsc.subcore_barrier()

        pltpu.async_copy(feat.at[src_v.at[0]], b0, sem0)

        def step(j):
            for bufc, semc, bufn, semn, t in ((b0, sem0, b1, sem1, 0),
                                              (b1, sem1, b0, sem0, 1)):
                cur = j + t
                pltpu.async_copy(feat.at[src_v.at[cur + 1]], bufn, semn)
                pltpu.make_async_copy(feat.at[src_v.at[cur]], bufc, semc).wait()
                pltpu.sync_copy(bufc, acc.at[dst_v.at[cur]], add=True)

        pl.loop(0, nch, step=2)(step)
        pltpu.make_async_copy(feat.at[src_v.at[0]], b0, sem0).wait()
        plsc.subcore_barrier()

        for m in range(rows_sub // CH):
            pltpu.sync_copy(acc.at[pl.ds(off + m * CH, CH)], b0)
            pltpu.sync_copy(b0, out.at[c, pl.ds(off + m * CH, CH)])

    return pl.kernel(body, mesh=mesh, out_type=out_types,
                     scratch_types=scratch,
                     compiler_params=pltpu.CompilerParams(
                         use_tc_tiling_on_sc=False))


def _build_deg_kernel(np_, nacc, nch):
    mesh = plsc.VectorSubcoreMesh(core_axis_name="c", subcore_axis_name="s")
    rows_sub = nacc // NS
    out_types = jax.ShapeDtypeStruct((NC, np_, 16), jnp.float32)
    scratch = [
        pltpu.VMEM((nch, CH), jnp.int32),
        pltpu.VMEM((CH, 16), jnp.float32),
        pltpu.VMEM((CH, 16), jnp.float32),
        pltpu.VMEM_SHARED((nacc, 16), jnp.float32),
    ]

    def body(dstp, out, dst_v, ones_v, z16, degtab):
        c = lax.axis_index("c")
        s = lax.axis_index("s")
        pltpu.sync_copy(dstp.at[s], dst_v)

        zero16 = jnp.zeros((16,), jnp.float32)
        one16 = jnp.ones((16,), jnp.float32)

        def zf(i, carry):
            ones_v[i, pl.ds(0, 16)] = one16
            z16[i, pl.ds(0, 16)] = zero16
            return carry
        lax.fori_loop(0, CH, zf, 0)

        off = s * rows_sub
        nfull, rem = rows_sub // CH, rows_sub % CH
        for m in range(nfull):
            pltpu.sync_copy(z16, degtab.at[pl.ds(off + m * CH, CH)])
        if rem:
            pltpu.sync_copy(z16.at[pl.ds(0, rem)],
                            degtab.at[pl.ds(off + nfull * CH, rem)])
        plsc.subcore_barrier()

        def step(j):
            pltpu.sync_copy(ones_v, degtab.at[dst_v.at[j]], add=True)
        pl.loop(0, nch, step=1)(step)
        plsc.subcore_barrier()

        for m in range(rows_sub // CH):
            pltpu.sync_copy(degtab.at[pl.ds(off + m * CH, CH)], z16)
            pltpu.sync_copy(z16, out.at[c, pl.ds(off + m * CH, CH)])

    return pl.kernel(body, mesh=mesh, out_type=out_types,
                     scratch_types=scratch,
                     compiler_params=pltpu.CompilerParams(
                         use_tc_tiling_on_sc=False))


def _prep_edges(src, dst, n, nch):
    epad = NS * nch * CH
    e = src.shape[0]
    fill = jnp.full((epad - e,), n, jnp.int32)
    srcp = (2 * jnp.concatenate([src, fill])).reshape(NS, nch, CH)
    srcp = jnp.concatenate(
        [srcp, jnp.full((NS, 1, CH), 2 * n, jnp.int32)], axis=1)
    dstp = jnp.concatenate([dst, fill]).reshape(NS, nch, CH)
    return srcp, dstp


_BISECT = False
_BISECT_DEG = False


def _message(feat_pad, srcp, dstp, np_, nacc, nch):
    if _BISECT:
        src = srcp[:, :-1].reshape(-1) // 2
        dst = dstp.reshape(-1)
        agg = jax.ops.segment_sum(feat_pad[src], dst, num_segments=np_)
        return jnp.stack([agg[:, :HD], agg[:, HD:]])
    feat2 = feat_pad.reshape(2 * np_, HD)
    return _build_msg_kernel(np_, nacc, nch)(feat2, srcp, dstp)


def _degree(dstp, np_, nacc, nch):
    if _BISECT_DEG:
        return jnp.bincount(dstp.reshape(-1), length=np_).astype(
            jnp.float32)[:, None]
    degparts = _build_deg_kernel(np_, nacc, nch)(dstp)
    return degparts[0, :, 0:1]


def kernel(x, edge_index, batch, params):
    n, d = x.shape
    p = params
    np_ = ((n + MAXN + BK - 1) // BK) * BK
    src = edge_index[0].astype(jnp.int32)
    dst = edge_index[1].astype(jnp.int32)
    batch = batch.astype(jnp.int32)

    x_pad = jnp.zeros((np_, d), jnp.float32).at[:n].set(x)
    e = src.shape[0]
    nch = -(-e // (NS * CH))
    nch += nch % 2
    nacc = -(-(n + 1) // (NS * CH)) * (NS * CH)
    srcp, dstp = _prep_edges(src, dst, n, nch)

    bias = lambda k: p[k][None, :]

    t = _call_rows(
        _mlp_head_body, np_,
        [x_pad, p['in_W'], bias('in_b'), p['l1_W'], bias('l1_b'),
         p['l2_W'], bias('l2_b'), p['l3_W'], bias('l3_b')],
        ["rows", (D, D), (1, D), (D, D), (1, D), (D, D), (1, D), (D, D), (1, D)],
        n_row_outs=1)

    dg0 = _degree(dstp, np_, nacc, nch)
    for j in (1, 2, 3):
        parts = _message(t, srcp, dstp, np_, nacc, nch)
        p0, p1 = parts[0], parts[1]
        if j < 3:
            nw, nb = p[f'l{j + 1}_W'], bias(f'l{j + 1}_b')
            t = _call_rows(
                _combine_body, np_,
                [p0, p1, t, dg0, p[f'c{j}_Wl'], bias(f'c{j}_bl'),
                 p[f'c{j}_Wr'], nw, nb],
                ["half", "half", "rows", "col",
                 (D, D), (1, D), (D, D), (D, D), (1, D)],
                n_row_outs=1)
        else:
            yk, yv, c = _call_rows(
                _final_body, np_,
                [p0, p1, t, dg0, p['c3_Wl'], bias('c3_bl'), p['c3_Wr'],
                 p['p_lin1_W'], bias('p_lin1_b'), p['pma1_k_W'], p['pma1_v_W']],
                ["half", "half", "rows", "col",
                 (D, D), (1, D), (D, D), (D, D), (1, D), (D, D), (D, D)],
                n_row_outs=2, n_col_outs=1)

    kparts = _message(yk, srcp, dstp, np_, nacc, nch)
    vparts = _message(yv, srcp, dstp, np_, nacc, nch)
    kf, vf = _call_rows(
        _kv_finish_body, np_,
        [kparts[0], kparts[1], yk, vparts[0], vparts[1], yv, c,
         bias('pma1_k_b'), bias('pma1_v_b')],
        ["half", "half", "rows", "half", "half", "rows", "col",
         (1, D), (1, D)],
        n_row_outs=2)

    ptr = jnp.searchsorted(batch, jnp.arange(B, dtype=jnp.int32)).astype(jnp.int32)
    counts = jnp.diff(jnp.concatenate([ptr, jnp.array([n], jnp.int32)]))

    return _attention(counts, ptr, kf, vf, p)


# speedup vs baseline: 6.5338x; 1.0012x over previous
import functools
import math

import jax
import jax.numpy as jnp
from jax import lax
from jax.experimental import pallas as pl
from jax.experimental.pallas import tpu as pltpu
from jax.experimental.pallas import tpu_sc as plsc

D = 128
B = 40
MAXN = 512
H = 4
DH = D // H
K1 = 75
K1P = 80
WIN = MAXN + 8
BK = 512
NEG = -1e9
_INV_SQRT_D = 1.0 / math.sqrt(D)


def _relu(v):
    return jnp.maximum(v, 0.0)


def _mm(a, b):
    return jnp.dot(a, b, preferred_element_type=jnp.float32)


def _mlp_head_body(x_ref, w0, b0, w1, b1, w2, b2, w3, b3, t_ref):
    h = x_ref[...]
    for w, b in ((w0, b0), (w1, b1), (w2, b2), (w3, b3)):
        h = _relu(_mm(h, w[...]) + b[...])
    t_ref[...] = _relu(_mm(h, w1[...]) + b1[...])


def _combine_body(p0, p1, t, dg0, wl, bl, wr, wn, bn, t_ref):
    deg = dg0[...]
    agg = jnp.concatenate([p0[...], p1[...]], axis=1) \
        * (1.0 / jnp.maximum(deg, 1.0))
    u = _relu(_mm(agg, wl[...]) + bl[...] + _mm(t[...], wr[...]))
    t_ref[...] = _relu(_mm(u, wn[...]) + bn[...])


def _final_body(p0, p1, t, dg0, wl, bl, wr, pw, pb, kw, vw,
                yk_ref, yv_ref, c_ref):
    deg = dg0[...]
    agg = jnp.concatenate([p0[...], p1[...]], axis=1) \
        * (1.0 / jnp.maximum(deg, 1.0))
    u = _relu(_mm(agg, wl[...]) + bl[...] + _mm(t[...], wr[...]))
    xf = _mm(u, pw[...]) + pb[...]
    c = lax.rsqrt(deg + 1.0)
    yk_ref[...] = _mm(xf, kw[...]) * c
    yv_ref[...] = _mm(xf, vw[...]) * c
    c_ref[...] = c


def _kv_finish_body(pk0, pk1, yk, pv0, pv1, yv, c, bk, bv, kf_ref, vf_ref):
    cc = c[...]
    pk = jnp.concatenate([pk0[...], pk1[...]], axis=1)
    pv = jnp.concatenate([pv0[...], pv1[...]], axis=1)
    kf_ref[...] = (pk + yk[...]) * cc + bk[...]
    vf_ref[...] = (pv + yv[...]) * cc + bv[...]


def _row_specs(n, shapes):
    specs = []
    for s in shapes:
        if s == "rows":
            specs.append(pl.BlockSpec((BK, D), lambda i: (i, 0)))
        elif s == "col":
            specs.append(pl.BlockSpec((BK, 1), lambda i: (i, 0)))
        elif s == "half":
            specs.append(pl.BlockSpec((BK, HD), lambda i: (i, 0)))
        else:
            specs.append(pl.BlockSpec(s, lambda i: (0,) * len(s)))
    return specs


def _call_rows(body, np_, in_arrays, in_kinds, n_row_outs, n_col_outs=0):
    grid = (np_ // BK,)
    out_shapes = ([jax.ShapeDtypeStruct((np_, D), jnp.float32)] * n_row_outs
                  + [jax.ShapeDtypeStruct((np_, 1), jnp.float32)] * n_col_outs)
    out_specs = ([pl.BlockSpec((BK, D), lambda i: (i, 0))] * n_row_outs
                 + [pl.BlockSpec((BK, 1), lambda i: (i, 0))] * n_col_outs)
    return pl.pallas_call(
        body,
        grid=grid,
        in_specs=_row_specs(np_, in_kinds),
        out_specs=out_specs if len(out_specs) > 1 else out_specs[0],
        out_shape=out_shapes if len(out_shapes) > 1 else out_shapes[0],
    )(*in_arrays)


def _mab(q_in, kd, vd, wq, bq, wo, bo, maskrow):
    qp = _mm(q_in, wq) + bq
    outs = []
    for h in range(H):
        q = qp[:, h * DH:(h + 1) * DH]
        k = kd[:, h * DH:(h + 1) * DH]
        v = vd[:, h * DH:(h + 1) * DH]
        att = lax.dot_general(q, k, (((1,), (1,)), ((), ()))) * _INV_SQRT_D
        att = att + maskrow
        m = jnp.max(att, axis=-1, keepdims=True)
        e = jnp.exp(att - m)
        a = e / jnp.sum(e, axis=-1, keepdims=True)
        outs.append(q + _mm(a, v))
    o = jnp.concatenate(outs, axis=1)
    return o + _relu(_mm(o, wo) + bo)


def _attn_body(count_ref, ptr_ref, kf_hbm, vf_hbm, s1, fcqw, fcqb, ow, ob,
               skw, skb, svw, svb, sfw, sfb, sow, sob,
               pkw, pkb, pvw, pvb, pfw, pfb, pow_, pob, s2, w2, b2,
               out_ref, kbuf, vbuf, sem0, sem1):
    b = pl.program_id(0)
    cnt = jnp.minimum(count_ref[b], MAXN)
    start = ptr_ref[b]
    start8 = (start // 8) * 8
    shift = start - start8
    cpk = pltpu.make_async_copy(kf_hbm.at[pl.ds(start8, WIN)], kbuf, sem0)
    cpv = pltpu.make_async_copy(vf_hbm.at[pl.ds(start8, WIN)], vbuf, sem1)
    cpk.start()
    cpv.start()
    cpk.wait()
    cpv.wait()
    col = lax.broadcasted_iota(jnp.int32, (1, WIN), 1)
    mask1 = jnp.where((col >= shift) & (col < shift + cnt), 0.0, NEG)
    rowi = lax.broadcasted_iota(jnp.int32, (WIN, 1), 0)
    rvalid = (rowi >= shift) & (rowi < shift + cnt)
    kd = jnp.where(rvalid, kbuf[...], 0.0)
    vd = jnp.where(rvalid, vbuf[...], 0.0)

    bx = _mab(s1[...], kd, vd, fcqw[...], fcqb[...], ow[...], ob[...], mask1)

    colp = lax.broadcasted_iota(jnp.int32, (1, K1P), 1)
    maskp = jnp.where(colp >= K1, NEG, 0.0)
    kd2 = _mm(bx, skw[...]) + skb[...]
    vd2 = _mm(bx, svw[...]) + svb[...]
    bx = _mab(bx, kd2, vd2, sfw[...], sfb[...], sow[...], sob[...], maskp)

    kd3 = _mm(bx, pkw[...]) + pkb[...]
    vd3 = _mm(bx, pvw[...]) + pvb[...]
    bx3 = _mab(s2[...], kd3, vd3, pfw[...], pfb[...], pow_[...], pob[...], maskp)

    res = jnp.sum(bx3[0:1, :] * w2[...]) + b2[0, 0]
    out_ref[...] = jnp.broadcast_to(res, (1, 8, D))


def _attention(counts, ptr, kf, vf, p):
    s1 = jnp.zeros((K1P, D), jnp.float32).at[:K1].set(p['pma1_S'])
    s2 = jnp.zeros((8, D), jnp.float32).at[0:1].set(p['pma2_S'])
    w2 = p['p_lin2_W'].T
    b2 = p['p_lin2_b'].reshape(1, 1)

    def wspec(shape):
        return pl.BlockSpec(shape, lambda b, c, q: (0,) * len(shape))

    ins = [kf, vf, s1,
           p['pma1_fcq_W'], p['pma1_fcq_b'][None, :],
           p['pma1_o_W'], p['pma1_o_b'][None, :],
           p['sab_k_W'], p['sab_k_b'][None, :],
           p['sab_v_W'], p['sab_v_b'][None, :],
           p['sab_fcq_W'], p['sab_fcq_b'][None, :],
           p['sab_o_W'], p['sab_o_b'][None, :],
           p['pma2_k_W'], p['pma2_k_b'][None, :],
           p['pma2_v_W'], p['pma2_v_b'][None, :],
           p['pma2_fcq_W'], p['pma2_fcq_b'][None, :],
           p['pma2_o_W'], p['pma2_o_b'][None, :],
           s2, w2, b2]
    in_specs = [pl.BlockSpec(memory_space=pl.ANY),
                pl.BlockSpec(memory_space=pl.ANY)]
    for a in ins[2:]:
        in_specs.append(wspec(a.shape))

    grid_spec = pltpu.PrefetchScalarGridSpec(
        num_scalar_prefetch=2,
        grid=(B,),
        in_specs=in_specs,
        out_specs=pl.BlockSpec((1, 8, D), lambda b, c, q: (b, 0, 0)),
        scratch_shapes=[pltpu.VMEM((WIN, D), jnp.float32),
                        pltpu.VMEM((WIN, D), jnp.float32),
                        pltpu.SemaphoreType.DMA,
                        pltpu.SemaphoreType.DMA],
    )
    out = pl.pallas_call(
        _attn_body,
        grid_spec=grid_spec,
        out_shape=jax.ShapeDtypeStruct((B, 8, D), jnp.float32),
    )(counts, ptr, *ins)
    return out[:, 0, 0]


NC = 2
NS = 16
TILES = NC * NS
CH = 128


HD = D // 2


def _build_msg_kernel(np_, nacc, nch):
    mesh = plsc.VectorSubcoreMesh(core_axis_name="c", subcore_axis_name="s")
    rows_sub = nacc // NS
    out_types = jax.ShapeDtypeStruct((NC, np_, HD), jnp.float32)
    scratch = [
        pltpu.VMEM((nch + 1, CH), jnp.int32),
        pltpu.VMEM((nch, CH), jnp.int32),
        pltpu.VMEM((CH, HD), jnp.float32),
        pltpu.VMEM((CH, HD), jnp.float32),
        pltpu.VMEM_SHARED((nacc, HD), jnp.float32),
        pltpu.SemaphoreType.DMA,
        pltpu.SemaphoreType.DMA,
    ]

    def body(feat, srcp, dstp, out, src_v, dst_v, b0, b1, acc, sem0, sem1):
        c = lax.axis_index("c")
        s = lax.axis_index("s")
        pltpu.sync_copy(srcp.at[s], src_v)
        pltpu.sync_copy(dstp.at[s], dst_v)

        zero16 = jnp.zeros((16,), jnp.float32)

        @pl.when(c == 1)
        def _():
            def addc(i, carry):
                for k in range(CH // 16):
                    sl = pl.ds(k * 16, 16)
                    src_v[i, sl] = src_v[i, sl] + 1
                return carry
            lax.fori_loop(0, nch + 1, addc, 0)

        def zf(i, carry):
            for k in range(HD // 16):
                b0[i, pl.ds(k * 16, 16)] = zero16
            return carry
        lax.fori_loop(0, CH, zf, 0)

        off = s * rows_sub
        nfull, rem = rows_sub // CH, rows_sub % CH
        for m in range(nfull):
            pltpu.sync_copy(b0, acc.at[pl.ds(off + m * CH, CH)])
        if rem:
            pltpu.sync_copy(b0.at[pl.ds(0, rem)],
                            acc.at[pl.ds(off + nfull * CH, rem)])
        plsc.subcore_barrier()

        pltpu.async_copy(feat.at[src_v.at[0]], b0, sem0)

        def step(j):
            for bufc, semc, bufn, semn, t in ((b0, sem0, b1, sem1, 0),
                                              (b1, sem1, b0, sem0, 1)):
                cur = j + t
                pltpu.async_copy(feat.at[src_v.at[cur + 1]], bufn, semn)
                pltpu.make_async_copy(feat.at[src_v.at[cur]], bufc, semc).wait()
                pltpu.sync_copy(bufc, acc.at[dst_v.at[cur]], add=True)

        pl.loop(0, nch, step=2)(step)
        pltpu.make_async_copy(feat.at[src_v.at[0]], b0, sem0).wait()
        plsc.subcore_barrier()

        for m in range(rows_sub // CH):
            pltpu.sync_copy(acc.at[pl.ds(off + m * CH, CH)], b0)
            pltpu.sync_copy(b0, out.at[c, pl.ds(off + m * CH, CH)])

    return pl.kernel(body, mesh=mesh, out_type=out_types,
                     scratch_types=scratch,
                     compiler_params=pltpu.CompilerParams(
                         use_tc_tiling_on_sc=False))


def _build_deg_kernel(np_, nacc, nch):
    mesh = plsc.VectorSubcoreMesh(core_axis_name="c", subcore_axis_name="s")
    rows_sub = nacc // NS
    out_types = jax.ShapeDtypeStruct((NC, np_, 16), jnp.float32)
    scratch = [
        pltpu.VMEM((nch, CH), jnp.int32),
        pltpu.VMEM((CH, 16), jnp.float32),
        pltpu.VMEM((CH, 16), jnp.float32),
        pltpu.VMEM_SHARED((nacc, 16), jnp.float32),
    ]

    def body(dstp, out, dst_v, ones_v, z16, degtab):
        c = lax.axis_index("c")
        s = lax.axis_index("s")
        pltpu.sync_copy(dstp.at[s], dst_v)

        zero16 = jnp.zeros((16,), jnp.float32)
        one16 = jnp.ones((16,), jnp.float32)

        def zf(i, carry):
            ones_v[i, pl.ds(0, 16)] = one16
            z16[i, pl.ds(0, 16)] = zero16
            return carry
        lax.fori_loop(0, CH, zf, 0)

        off = s * rows_sub
        nfull, rem = rows_sub // CH, rows_sub % CH
        for m in range(nfull):
            pltpu.sync_copy(z16, degtab.at[pl.ds(off + m * CH, CH)])
        if rem:
            pltpu.sync_copy(z16.at[pl.ds(0, rem)],
                            degtab.at[pl.ds(off + nfull * CH, rem)])
        plsc.subcore_barrier()

        def step(j):
            pltpu.sync_copy(ones_v, degtab.at[dst_v.at[j]], add=True)
        pl.loop(0, nch, step=1)(step)
        plsc.subcore_barrier()

        for m in range(rows_sub // CH):
            pltpu.sync_copy(degtab.at[pl.ds(off + m * CH, CH)], z16)
            pltpu.sync_copy(z16, out.at[c, pl.ds(off + m * CH, CH)])

    return pl.kernel(body, mesh=mesh, out_type=out_types,
                     scratch_types=scratch,
                     compiler_params=pltpu.CompilerParams(
                         use_tc_tiling_on_sc=False))


def _prep_edges(src, dst, n, nch):
    epad = NS * nch * CH
    e = src.shape[0]
    fill = jnp.full((epad - e,), n, jnp.int32)
    srcp = (2 * jnp.concatenate([src, fill])).reshape(NS, nch, CH)
    srcp = jnp.concatenate(
        [srcp, jnp.full((NS, 1, CH), 2 * n, jnp.int32)], axis=1)
    dstp = jnp.concatenate([dst, fill]).reshape(NS, nch, CH)
    return srcp, dstp


def _message(feat_pad, srcp, dstp, np_, nacc, nch):
    feat2 = feat_pad.reshape(2 * np_, HD)
    return _build_msg_kernel(np_, nacc, nch)(feat2, srcp, dstp)


def _degree(dstp, np_, nacc, nch):
    degparts = _build_deg_kernel(np_, nacc, nch)(dstp)
    return degparts[0, :, 0:1]


def kernel(x, edge_index, batch, params):
    n, d = x.shape
    p = params
    np_ = ((n + MAXN + BK - 1) // BK) * BK
    src = edge_index[0].astype(jnp.int32)
    dst = edge_index[1].astype(jnp.int32)
    batch = batch.astype(jnp.int32)

    x_pad = jnp.zeros((np_, d), jnp.float32).at[:n].set(x)
    e = src.shape[0]
    nch = -(-e // (NS * CH))
    nch += nch % 2
    nacc = -(-(n + 1) // (NS * CH)) * (NS * CH)
    srcp, dstp = _prep_edges(src, dst, n, nch)

    bias = lambda k: p[k][None, :]

    t = _call_rows(
        _mlp_head_body, np_,
        [x_pad, p['in_W'], bias('in_b'), p['l1_W'], bias('l1_b'),
         p['l2_W'], bias('l2_b'), p['l3_W'], bias('l3_b')],
        ["rows", (D, D), (1, D), (D, D), (1, D), (D, D), (1, D), (D, D), (1, D)],
        n_row_outs=1)

    dg0 = _degree(dstp, np_, nacc, nch)
    for j in (1, 2, 3):
        parts = _message(t, srcp, dstp, np_, nacc, nch)
        p0, p1 = parts[0], parts[1]
        if j < 3:
            nw, nb = p[f'l{j + 1}_W'], bias(f'l{j + 1}_b')
            t = _call_rows(
                _combine_body, np_,
                [p0, p1, t, dg0, p[f'c{j}_Wl'], bias(f'c{j}_bl'),
                 p[f'c{j}_Wr'], nw, nb],
                ["half", "half", "rows", "col",
                 (D, D), (1, D), (D, D), (D, D), (1, D)],
                n_row_outs=1)
        else:
            yk, yv, c = _call_rows(
                _final_body, np_,
                [p0, p1, t, dg0, p['c3_Wl'], bias('c3_bl'), p['c3_Wr'],
                 p['p_lin1_W'], bias('p_lin1_b'), p['pma1_k_W'], p['pma1_v_W']],
                ["half", "half", "rows", "col",
                 (D, D), (1, D), (D, D), (D, D), (1, D), (D, D), (D, D)],
                n_row_outs=2, n_col_outs=1)

    kparts = _message(yk, srcp, dstp, np_, nacc, nch)
    vparts = _message(yv, srcp, dstp, np_, nacc, nch)
    kf, vf = _call_rows(
        _kv_finish_body, np_,
        [kparts[0], kparts[1], yk, vparts[0], vparts[1], yv, c,
         bias('pma1_k_b'), bias('pma1_v_b')],
        ["half", "half", "rows", "half", "half", "rows", "col",
         (1, D), (1, D)],
        n_row_outs=2)

    ptr = jnp.searchsorted(batch, jnp.arange(B, dtype=jnp.int32)).astype(jnp.int32)
    counts = jnp.diff(jnp.concatenate([ptr, jnp.array([n], jnp.int32)]))

    return _attention(counts, ptr, kf, vf, p)


# speedup vs baseline: 6.5633x; 1.0045x over previous
import functools
import math

import jax
import jax.numpy as jnp
from jax import lax
from jax.experimental import pallas as pl
from jax.experimental.pallas import tpu as pltpu
from jax.experimental.pallas import tpu_sc as plsc

D = 128
B = 40
MAXN = 512
H = 4
DH = D // H
K1 = 75
K1P = 80
WIN = MAXN + 8
BK = 512
NEG = -1e9
_INV_SQRT_D = 1.0 / math.sqrt(D)


def _relu(v):
    return jnp.maximum(v, 0.0)


def _mm(a, b):
    return jnp.dot(a, b, preferred_element_type=jnp.float32)


def _mlp_head_body(x_ref, w0, b0, w1, b1, w2, b2, w3, b3, t_ref):
    h = x_ref[...]
    for w, b in ((w0, b0), (w1, b1), (w2, b2), (w3, b3)):
        h = _relu(_mm(h, w[...]) + b[...])
    t_ref[...] = _relu(_mm(h, w1[...]) + b1[...])


def _combine_body(p0, p1, t, dg0, wl, bl, wr, wn, bn, t_ref):
    deg = dg0[...]
    agg = jnp.concatenate([p0[...], p1[...]], axis=1) \
        * (1.0 / jnp.maximum(deg, 1.0))
    u = _relu(_mm(agg, wl[...]) + bl[...] + _mm(t[...], wr[...]))
    t_ref[...] = _relu(_mm(u, wn[...]) + bn[...])


def _final_body(p0, p1, t, dg0, wl, bl, wr, pw, pb, kw, vw,
                yk_ref, yv_ref, c_ref):
    deg = dg0[...]
    agg = jnp.concatenate([p0[...], p1[...]], axis=1) \
        * (1.0 / jnp.maximum(deg, 1.0))
    u = _relu(_mm(agg, wl[...]) + bl[...] + _mm(t[...], wr[...]))
    xf = _mm(u, pw[...]) + pb[...]
    c = lax.rsqrt(deg + 1.0)
    yk_ref[...] = _mm(xf, kw[...]) * c
    yv_ref[...] = _mm(xf, vw[...]) * c
    c_ref[...] = c


def _kv_finish_body(pk0, pk1, yk, pv0, pv1, yv, c, bk, bv, kf_ref, vf_ref):
    cc = c[...]
    pk = jnp.concatenate([pk0[...], pk1[...]], axis=1)
    pv = jnp.concatenate([pv0[...], pv1[...]], axis=1)
    kf_ref[...] = (pk + yk[...]) * cc + bk[...]
    vf_ref[...] = (pv + yv[...]) * cc + bv[...]


def _row_specs(n, shapes):
    specs = []
    for s in shapes:
        if s == "rows":
            specs.append(pl.BlockSpec((BK, D), lambda i: (i, 0)))
        elif s == "col":
            specs.append(pl.BlockSpec((BK, 1), lambda i: (i, 0)))
        elif s == "half":
            specs.append(pl.BlockSpec((BK, HD), lambda i: (i, 0)))
        else:
            specs.append(pl.BlockSpec(s, lambda i: (0,) * len(s)))
    return specs


def _call_rows(body, np_, in_arrays, in_kinds, n_row_outs, n_col_outs=0):
    grid = (np_ // BK,)
    out_shapes = ([jax.ShapeDtypeStruct((np_, D), jnp.float32)] * n_row_outs
                  + [jax.ShapeDtypeStruct((np_, 1), jnp.float32)] * n_col_outs)
    out_specs = ([pl.BlockSpec((BK, D), lambda i: (i, 0))] * n_row_outs
                 + [pl.BlockSpec((BK, 1), lambda i: (i, 0))] * n_col_outs)
    return pl.pallas_call(
        body,
        grid=grid,
        in_specs=_row_specs(np_, in_kinds),
        out_specs=out_specs if len(out_specs) > 1 else out_specs[0],
        out_shape=out_shapes if len(out_shapes) > 1 else out_shapes[0],
    )(*in_arrays)


def _mab(q_in, kd, vd, wq, bq, wo, bo, maskrow):
    qp = _mm(q_in, wq) + bq
    outs = []
    for h in range(H):
        q = qp[:, h * DH:(h + 1) * DH]
        k = kd[:, h * DH:(h + 1) * DH]
        v = vd[:, h * DH:(h + 1) * DH]
        att = lax.dot_general(q, k, (((1,), (1,)), ((), ()))) * _INV_SQRT_D
        att = att + maskrow
        m = jnp.max(att, axis=-1, keepdims=True)
        e = jnp.exp(att - m)
        a = e / jnp.sum(e, axis=-1, keepdims=True)
        outs.append(q + _mm(a, v))
    o = jnp.concatenate(outs, axis=1)
    return o + _relu(_mm(o, wo) + bo)


def _attn_body(count_ref, ptr_ref, kf_hbm, vf_hbm, s1, fcqw, fcqb, ow, ob,
               skw, skb, svw, svb, sfw, sfb, sow, sob,
               pkw, pkb, pvw, pvb, pfw, pfb, pow_, pob, s2, w2, b2,
               out_ref, kbuf, vbuf, sem0, sem1):
    b = pl.program_id(0)
    cnt = jnp.minimum(count_ref[b], MAXN)
    start = ptr_ref[b]
    start8 = (start // 8) * 8
    shift = start - start8
    cpk = pltpu.make_async_copy(kf_hbm.at[pl.ds(start8, WIN)], kbuf, sem0)
    cpv = pltpu.make_async_copy(vf_hbm.at[pl.ds(start8, WIN)], vbuf, sem1)
    cpk.start()
    cpv.start()
    cpk.wait()
    cpv.wait()
    col = lax.broadcasted_iota(jnp.int32, (1, WIN), 1)
    mask1 = jnp.where((col >= shift) & (col < shift + cnt), 0.0, NEG)
    rowi = lax.broadcasted_iota(jnp.int32, (WIN, 1), 0)
    rvalid = (rowi >= shift) & (rowi < shift + cnt)
    kd = jnp.where(rvalid, kbuf[...], 0.0)
    vd = jnp.where(rvalid, vbuf[...], 0.0)

    bx = _mab(s1[...], kd, vd, fcqw[...], fcqb[...], ow[...], ob[...], mask1)

    colp = lax.broadcasted_iota(jnp.int32, (1, K1P), 1)
    maskp = jnp.where(colp >= K1, NEG, 0.0)
    kd2 = _mm(bx, skw[...]) + skb[...]
    vd2 = _mm(bx, svw[...]) + svb[...]
    bx = _mab(bx, kd2, vd2, sfw[...], sfb[...], sow[...], sob[...], maskp)

    kd3 = _mm(bx, pkw[...]) + pkb[...]
    vd3 = _mm(bx, pvw[...]) + pvb[...]
    bx3 = _mab(s2[...], kd3, vd3, pfw[...], pfb[...], pow_[...], pob[...], maskp)

    res = jnp.sum(bx3[0:1, :] * w2[...]) + b2[0, 0]
    out_ref[...] = jnp.broadcast_to(res, (1, 8, D))


def _attention(counts, ptr, kf, vf, p):
    s1 = jnp.zeros((K1P, D), jnp.float32).at[:K1].set(p['pma1_S'])
    s2 = jnp.zeros((8, D), jnp.float32).at[0:1].set(p['pma2_S'])
    w2 = p['p_lin2_W'].T
    b2 = p['p_lin2_b'].reshape(1, 1)

    def wspec(shape):
        return pl.BlockSpec(shape, lambda b, c, q: (0,) * len(shape))

    ins = [kf, vf, s1,
           p['pma1_fcq_W'], p['pma1_fcq_b'][None, :],
           p['pma1_o_W'], p['pma1_o_b'][None, :],
           p['sab_k_W'], p['sab_k_b'][None, :],
           p['sab_v_W'], p['sab_v_b'][None, :],
           p['sab_fcq_W'], p['sab_fcq_b'][None, :],
           p['sab_o_W'], p['sab_o_b'][None, :],
           p['pma2_k_W'], p['pma2_k_b'][None, :],
           p['pma2_v_W'], p['pma2_v_b'][None, :],
           p['pma2_fcq_W'], p['pma2_fcq_b'][None, :],
           p['pma2_o_W'], p['pma2_o_b'][None, :],
           s2, w2, b2]
    in_specs = [pl.BlockSpec(memory_space=pl.ANY),
                pl.BlockSpec(memory_space=pl.ANY)]
    for a in ins[2:]:
        in_specs.append(wspec(a.shape))

    grid_spec = pltpu.PrefetchScalarGridSpec(
        num_scalar_prefetch=2,
        grid=(B,),
        in_specs=in_specs,
        out_specs=pl.BlockSpec((1, 8, D), lambda b, c, q: (b, 0, 0)),
        scratch_shapes=[pltpu.VMEM((WIN, D), jnp.float32),
                        pltpu.VMEM((WIN, D), jnp.float32),
                        pltpu.SemaphoreType.DMA,
                        pltpu.SemaphoreType.DMA],
    )
    out = pl.pallas_call(
        _attn_body,
        grid_spec=grid_spec,
        out_shape=jax.ShapeDtypeStruct((B, 8, D), jnp.float32),
    )(counts, ptr, *ins)
    return out[:, 0, 0]


NC = 2
NS = 16
TILES = NC * NS
CH = 128


HD = D // 2


def _build_msg_kernel(np_, nacc, nch, with_deg=False):
    mesh = plsc.VectorSubcoreMesh(core_axis_name="c", subcore_axis_name="s")
    rows_sub = nacc // NS
    out_types = [jax.ShapeDtypeStruct((NC, np_, HD), jnp.float32)]
    scratch = [
        pltpu.VMEM((nch + 1, CH), jnp.int32),
        pltpu.VMEM((nch, CH), jnp.int32),
        pltpu.VMEM((CH, HD), jnp.float32),
        pltpu.VMEM((CH, HD), jnp.float32),
        pltpu.VMEM_SHARED((nacc, HD), jnp.float32),
        pltpu.SemaphoreType.DMA,
        pltpu.SemaphoreType.DMA,
    ]
    if with_deg:
        out_types.append(jax.ShapeDtypeStruct((NC, np_, 16), jnp.float32))
        scratch.append(pltpu.VMEM((CH, 16), jnp.float32))
        scratch.append(pltpu.VMEM((CH, 16), jnp.float32))
        scratch.append(pltpu.VMEM_SHARED((nacc, 16), jnp.float32))

    def body(feat, srcp, dstp, *rest):
        if with_deg:
            (out, dout, src_v, dst_v, b0, b1, acc, sem0, sem1,
             ones_v, z16, degtab) = rest
        else:
            out, src_v, dst_v, b0, b1, acc, sem0, sem1 = rest
        c = lax.axis_index("c")
        s = lax.axis_index("s")
        pltpu.sync_copy(srcp.at[s], src_v)
        pltpu.sync_copy(dstp.at[s], dst_v)

        zero16 = jnp.zeros((16,), jnp.float32)
        one16 = jnp.ones((16,), jnp.float32)

        @pl.when(c == 1)
        def _():
            def addc(i, carry):
                for k in range(CH // 16):
                    sl = pl.ds(k * 16, 16)
                    src_v[i, sl] = src_v[i, sl] + 1
                return carry
            lax.fori_loop(0, nch + 1, addc, 0)

        def zf(i, carry):
            for k in range(HD // 16):
                b0[i, pl.ds(k * 16, 16)] = zero16
            if with_deg:
                ones_v[i, pl.ds(0, 16)] = one16
                z16[i, pl.ds(0, 16)] = zero16
            return carry
        lax.fori_loop(0, CH, zf, 0)

        off = s * rows_sub
        nfull, rem = rows_sub // CH, rows_sub % CH
        for m in range(nfull):
            pltpu.sync_copy(b0, acc.at[pl.ds(off + m * CH, CH)])
            if with_deg:
                pltpu.sync_copy(z16, degtab.at[pl.ds(off + m * CH, CH)])
        if rem:
            pltpu.sync_copy(b0.at[pl.ds(0, rem)],
                            acc.at[pl.ds(off + nfull * CH, rem)])
            if with_deg:
                pltpu.sync_copy(z16.at[pl.ds(0, rem)],
                                degtab.at[pl.ds(off + nfull * CH, rem)])
        plsc.subcore_barrier()

        pltpu.async_copy(feat.at[src_v.at[0]], b0, sem0)

        def step(j):
            for bufc, semc, bufn, semn, t in ((b0, sem0, b1, sem1, 0),
                                              (b1, sem1, b0, sem0, 1)):
                cur = j + t
                pltpu.async_copy(feat.at[src_v.at[cur + 1]], bufn, semn)
                if with_deg:
                    pltpu.sync_copy(ones_v, degtab.at[dst_v.at[cur]],
                                    add=True)
                pltpu.make_async_copy(feat.at[src_v.at[cur]], bufc, semc).wait()
                pltpu.sync_copy(bufc, acc.at[dst_v.at[cur]], add=True)

        pl.loop(0, nch, step=2)(step)
        pltpu.make_async_copy(feat.at[src_v.at[0]], b0, sem0).wait()
        plsc.subcore_barrier()

        for m in range(rows_sub // CH):
            pltpu.sync_copy(acc.at[pl.ds(off + m * CH, CH)], b0)
            pltpu.sync_copy(b0, out.at[c, pl.ds(off + m * CH, CH)])
            if with_deg:
                pltpu.sync_copy(degtab.at[pl.ds(off + m * CH, CH)], z16)
                pltpu.sync_copy(z16, dout.at[c, pl.ds(off + m * CH, CH)])

    return pl.kernel(body, mesh=mesh,
                     out_type=out_types if with_deg else out_types[0],
                     scratch_types=scratch,
                     compiler_params=pltpu.CompilerParams(
                         use_tc_tiling_on_sc=False))


def _prep_edges(src, dst, n, nch):
    epad = NS * nch * CH
    e = src.shape[0]
    fill = jnp.full((epad - e,), n, jnp.int32)
    srcp = (2 * jnp.concatenate([src, fill])).reshape(NS, nch, CH)
    srcp = jnp.concatenate(
        [srcp, jnp.full((NS, 1, CH), 2 * n, jnp.int32)], axis=1)
    dstp = jnp.concatenate([dst, fill]).reshape(NS, nch, CH)
    return srcp, dstp


def _message(feat_pad, srcp, dstp, np_, nacc, nch, with_deg=False):
    feat2 = feat_pad.reshape(2 * np_, HD)
    return _build_msg_kernel(np_, nacc, nch, with_deg)(feat2, srcp, dstp)


def kernel(x, edge_index, batch, params):
    n, d = x.shape
    p = params
    np_ = ((n + MAXN + BK - 1) // BK) * BK
    src = edge_index[0].astype(jnp.int32)
    dst = edge_index[1].astype(jnp.int32)
    batch = batch.astype(jnp.int32)

    x_pad = jnp.zeros((np_, d), jnp.float32).at[:n].set(x)
    e = src.shape[0]
    nch = -(-e // (NS * CH))
    nch += nch % 2
    nacc = -(-(n + 1) // (NS * CH)) * (NS * CH)
    srcp, dstp = _prep_edges(src, dst, n, nch)

    bias = lambda k: p[k][None, :]

    t = _call_rows(
        _mlp_head_body, np_,
        [x_pad, p['in_W'], bias('in_b'), p['l1_W'], bias('l1_b'),
         p['l2_W'], bias('l2_b'), p['l3_W'], bias('l3_b')],
        ["rows", (D, D), (1, D), (D, D), (1, D), (D, D), (1, D), (D, D), (1, D)],
        n_row_outs=1)

    for j in (1, 2, 3):
        if j == 1:
            parts, degp = _message(t, srcp, dstp, np_, nacc, nch,
                                   with_deg=True)
            dg0 = degp[0, :, 0:1]
        else:
            parts = _message(t, srcp, dstp, np_, nacc, nch)
        p0, p1 = parts[0], parts[1]
        if j < 3:
            nw, nb = p[f'l{j + 1}_W'], bias(f'l{j + 1}_b')
            t = _call_rows(
                _combine_body, np_,
                [p0, p1, t, dg0, p[f'c{j}_Wl'], bias(f'c{j}_bl'),
                 p[f'c{j}_Wr'], nw, nb],
                ["half", "half", "rows", "col",
                 (D, D), (1, D), (D, D), (D, D), (1, D)],
                n_row_outs=1)
        else:
            yk, yv, c = _call_rows(
                _final_body, np_,
                [p0, p1, t, dg0, p['c3_Wl'], bias('c3_bl'), p['c3_Wr'],
                 p['p_lin1_W'], bias('p_lin1_b'), p['pma1_k_W'], p['pma1_v_W']],
                ["half", "half", "rows", "col",
                 (D, D), (1, D), (D, D), (D, D), (1, D), (D, D), (D, D)],
                n_row_outs=2, n_col_outs=1)

    kparts = _message(yk, srcp, dstp, np_, nacc, nch)
    vparts = _message(yv, srcp, dstp, np_, nacc, nch)
    kf, vf = _call_rows(
        _kv_finish_body, np_,
        [kparts[0], kparts[1], yk, vparts[0], vparts[1], yv, c,
         bias('pma1_k_b'), bias('pma1_v_b')],
        ["half", "half", "rows", "half", "half", "rows", "col",
         (1, D), (1, D)],
        n_row_outs=2)

    ptr = jnp.searchsorted(batch, jnp.arange(B, dtype=jnp.int32)).astype(jnp.int32)
    counts = jnp.diff(jnp.concatenate([ptr, jnp.array([n], jnp.int32)]))

    return _attention(counts, ptr, kf, vf, p)
